# baseline (speedup 1.0000x reference)
import functools

import jax
import jax.numpy as jnp
from jax import lax
from jax.experimental import pallas as pl
from jax.experimental.pallas import tpu as pltpu
from jax.experimental.pallas import tpu_sc as plsc

N = 10000
E = 320000
D = 128
C = 500
CP = 512
BLK = 1000
GRID = N // BLK

NC = 2
NS = 16
DH = D // NC
EPS = E // NS
B = 125
NBS = EPS // B
NBUF = 2
NRING = NBS // NBUF
NPAD = 10240
RPT = NPAD // NS
RB = 80
RZ = RPT // RB


def _tc_pre_body(x_ref, cid_ref, wmsg_ref, b_ref,
                 pl_ref, ph_ref, ql_ref, qh_ref, cl_ref, cnt_ref):
    i = pl.program_id(0)
    xb = x_ref[...]
    w1 = wmsg_ref[0:D, :]
    w2 = wmsg_ref[D:2 * D, :]
    wp = w1 + 0.5 * w2
    wq = 0.5 * w2
    p = jnp.dot(xb, wp, preferred_element_type=jnp.float32) + b_ref[...]
    q = jnp.dot(xb, wq, preferred_element_type=jnp.float32)
    pl_ref[...] = p[:, 0:DH].astype(jnp.bfloat16)
    ph_ref[...] = p[:, DH:D].astype(jnp.bfloat16)
    ql_ref[...] = q[:, 0:DH].astype(jnp.bfloat16)
    qh_ref[...] = q[:, DH:D].astype(jnp.bfloat16)
    oh = (cid_ref[...] == lax.broadcasted_iota(jnp.int32, (BLK, CP), 1)).astype(jnp.float32)
    clb = lax.dot_general(oh, xb, (((0,), (0,)), ((), ())),
                          preferred_element_type=jnp.float32)
    cntb = lax.dot_general(oh, jnp.ones((BLK, D), jnp.float32), (((0,), (0,)), ((), ())),
                           preferred_element_type=jnp.float32)

    @pl.when(i == 0)
    def _():
        cl_ref[...] = jnp.zeros_like(cl_ref)
        cnt_ref[...] = jnp.zeros_like(cnt_ref)

    cl_ref[...] += clb
    cnt_ref[...] += cntb


_tc_pre = pl.pallas_call(
    _tc_pre_body,
    grid=(GRID,),
    in_specs=[
        pl.BlockSpec((BLK, D), lambda i: (i, 0)),
        pl.BlockSpec((BLK, 1), lambda i: (i, 0)),
        pl.BlockSpec((2 * D, D), lambda i: (0, 0)),
        pl.BlockSpec((1, D), lambda i: (0, 0)),
    ],
    out_specs=[
        pl.BlockSpec((BLK, DH), lambda i: (i, 0)),
        pl.BlockSpec((BLK, DH), lambda i: (i, 0)),
        pl.BlockSpec((BLK, DH), lambda i: (i, 0)),
        pl.BlockSpec((BLK, DH), lambda i: (i, 0)),
        pl.BlockSpec((CP, D), lambda i: (0, 0)),
        pl.BlockSpec((CP, D), lambda i: (0, 0)),
    ],
    out_shape=[
        jax.ShapeDtypeStruct((N, DH), jnp.bfloat16),
        jax.ShapeDtypeStruct((N, DH), jnp.bfloat16),
        jax.ShapeDtypeStruct((N, DH), jnp.bfloat16),
        jax.ShapeDtypeStruct((N, DH), jnp.bfloat16),
        jax.ShapeDtypeStruct((CP, D), jnp.float32),
        jax.ShapeDtypeStruct((CP, D), jnp.float32),
    ],
)


_sc_mesh = plsc.VectorSubcoreMesh(core_axis_name="c", subcore_axis_name="s")


@functools.partial(
    pl.kernel,
    mesh=_sc_mesh,
    out_type=jax.ShapeDtypeStruct((NC, NPAD, DH), jnp.float32),
    scratch_types=(
        [pltpu.VMEM((NBS, B), jnp.int32),
         pltpu.VMEM((NBS, B), jnp.int32)]
        + [pltpu.VMEM((B, DH), jnp.bfloat16) for _ in range(2 * NBUF)]
        + [pltpu.VMEM((B, DH), jnp.float32) for _ in range(2)]
        + [pltpu.VMEM_SHARED((NPAD, DH), jnp.float32)]
        + [pltpu.SemaphoreType.DMA for _ in range(2 * NBUF + 2)]
    ),
    compiler_params=pltpu.CompilerParams(use_tc_tiling_on_sc=False),
)
def _sc_edge(pl_hbm, ph_hbm, ql_hbm, qh_hbm, src_hbm, dst_hbm, out_hbm,
             srcv, dstv,
             pb0, pb1, qb0, qb1, sb0, sb1, agg_sh,
             gp0, gp1, gq0, gq1, ss0, ss1):
    cid = lax.axis_index("c")
    sid = lax.axis_index("s")
    rb = sid * RPT
    pbs = (pb0, pb1)
    qbs = (qb0, qb1)
    sbs = (sb0, sb1)
    gps = (gp0, gp1)
    gqs = (gq0, gq1)
    sss = (ss0, ss1)

    pltpu.sync_copy(src_hbm.at[sid], srcv)
    pltpu.sync_copy(dst_hbm.at[sid], dstv)

    def _zrow(r, carry):
        for l in range(DH // 16):
            s = pl.ds(l * 16, 16)
            sb0[r, s] = jnp.zeros((16,), jnp.float32)
            sb1[r, s] = jnp.zeros((16,), jnp.float32)
        return carry

    lax.fori_loop(0, B, _zrow, 0)
    for z in range(RZ):
        pltpu.sync_copy(sb0.at[pl.ds(0, RB)], agg_sh.at[pl.ds(rb + z * RB, RB)])
    plsc.subcore_barrier()

    def _run(p_hbm, q_hbm):
        for k in range(2):
            pltpu.async_copy(sbs[k], agg_sh.at[dstv.at[k]], sss[k], add=True)
        for b in range(NBUF):
            pltpu.async_copy(p_hbm.at[srcv.at[b]], pbs[b], gps[b])
            pltpu.async_copy(q_hbm.at[dstv.at[b]], qbs[b], gqs[b])

        def _ring(g, carry):
            base = g * NBUF
            for b in range(NBUF):
                j = base + b
                k = b & 1
                pltpu.make_async_copy(p_hbm.at[srcv.at[j]], pbs[b], gps[b]).wait()
                pltpu.make_async_copy(q_hbm.at[dstv.at[j]], qbs[b], gqs[b]).wait()
                pltpu.make_async_copy(sbs[k], agg_sh.at[dstv.at[j]], sss[k]).wait()

                def _crow(r2, c2):
                    for u in range(5):
                        r = r2 * 5 + u
                        for l in range(DH // 32):
                            s = pl.ds(l * 32, 32)
                            v = jnp.maximum(pbs[b][r, s] + qbs[b][r, s],
                                            jnp.bfloat16(0.0))
                            sbs[k][r, s] = v.astype(jnp.float32)
                    return c2

                lax.fori_loop(0, B // 5, _crow, 0)
                pltpu.async_copy(sbs[k], agg_sh.at[dstv.at[j]], sss[k], add=True)
                j4 = j + NBUF
                jn = lax.select(j4 < NBS, j4, j4 - NBS)
                pltpu.async_copy(p_hbm.at[srcv.at[jn]], pbs[b], gps[b])
                pltpu.async_copy(q_hbm.at[dstv.at[jn]], qbs[b], gqs[b])
            return carry

        lax.fori_loop(0, NRING, _ring, 0)

        for k in range(2):
            pltpu.make_async_copy(sbs[k], agg_sh.at[dstv.at[k]], sss[k]).wait()
        for b in range(NBUF):
            pltpu.make_async_copy(p_hbm.at[srcv.at[b]], pbs[b], gps[b]).wait()
            pltpu.make_async_copy(q_hbm.at[dstv.at[b]], qbs[b], gqs[b]).wait()

    @pl.when(cid == 0)
    def _():
        _run(pl_hbm, ql_hbm)

    @pl.when(cid == 1)
    def _():
        _run(ph_hbm, qh_hbm)

    plsc.subcore_barrier()

    for z in range(RZ):
        pltpu.sync_copy(agg_sh.at[pl.ds(rb + z * RB, RB)], sb0.at[pl.ds(0, RB)])
        pltpu.sync_copy(sb0.at[pl.ds(0, RB)], out_hbm.at[cid, pl.ds(rb + z * RB, RB)])


def _tc_up_body(x_ref, cid_ref, cl_ref, cnt_ref, wup_ref, wu_ref, wc_ref, b_ref):
    xb = x_ref[...]
    up_attr = cl_ref[...] / jnp.maximum(cnt_ref[...], 1.0)
    msg_up = jnp.maximum(jnp.dot(up_attr, wup_ref[...], preferred_element_type=jnp.float32), 0.0)
    oh = (cid_ref[...] == lax.broadcasted_iota(jnp.int32, (BLK, CP), 1)).astype(jnp.float32)
    agg_up = jnp.dot(oh, msg_up, preferred_element_type=jnp.float32) + xb
    out_up = jnp.maximum(jnp.dot(agg_up, wu_ref[...], preferred_element_type=jnp.float32), 0.0)
    b_ref[...] = jnp.dot(out_up, wc_ref[D:2 * D, :], preferred_element_type=jnp.float32)


_tc_up = pl.pallas_call(
    _tc_up_body,
    grid=(GRID,),
    in_specs=[
        pl.BlockSpec((BLK, D), lambda i: (i, 0)),
        pl.BlockSpec((BLK, 1), lambda i: (i, 0)),
        pl.BlockSpec((CP, D), lambda i: (0, 0)),
        pl.BlockSpec((CP, D), lambda i: (0, 0)),
        pl.BlockSpec((D, D), lambda i: (0, 0)),
        pl.BlockSpec((D, D), lambda i: (0, 0)),
        pl.BlockSpec((2 * D, D), lambda i: (0, 0)),
    ],
    out_specs=pl.BlockSpec((BLK, D), lambda i: (i, 0)),
    out_shape=jax.ShapeDtypeStruct((N, D), jnp.float32),
)


def _tc_fin_body(x_ref, aggl_ref, aggh_ref, base_ref, wa_ref, wc_ref, o_ref):
    xb = x_ref[...]
    agg = jnp.concatenate([aggl_ref[...], aggh_ref[...]], axis=-1)
    adj = agg + xb
    out_adj = jnp.maximum(jnp.dot(adj, wa_ref[...], preferred_element_type=jnp.float32), 0.0)
    o_ref[...] = (jnp.dot(out_adj, wc_ref[0:D, :], preferred_element_type=jnp.float32)
                  + base_ref[...])


_tc_fin = pl.pallas_call(
    _tc_fin_body,
    grid=(GRID,),
    in_specs=[
        pl.BlockSpec((BLK, D), lambda i: (i, 0)),
        pl.BlockSpec((BLK, DH), lambda i: (i, 0)),
        pl.BlockSpec((BLK, DH), lambda i: (i, 0)),
        pl.BlockSpec((BLK, D), lambda i: (i, 0)),
        pl.BlockSpec((D, D), lambda i: (0, 0)),
        pl.BlockSpec((2 * D, D), lambda i: (0, 0)),
    ],
    out_specs=pl.BlockSpec((BLK, D), lambda i: (i, 0)),
    out_shape=jax.ShapeDtypeStruct((N, D), jnp.float32),
)


def kernel(x, edge_index, cluster_ids, W_msg, b_msg, W_up, W_upd_adj, W_upd_up, W_comb):
    src3 = edge_index[0].reshape(NS, NBS, B)
    dst3 = edge_index[1].reshape(NS, NBS, B)
    cid2 = cluster_ids.reshape(N, 1)
    bm = b_msg.reshape(1, D)
    PL, PH, QL, QH, cl, cnt = _tc_pre(x, cid2, W_msg, bm)
    parts = _sc_edge(PL, PH, QL, QH, src3, dst3)
    base = _tc_up(x, cid2, cl, cnt, W_up, W_upd_up, W_comb)
    out = _tc_fin(x, parts[0, :N], parts[1, :N], base, W_upd_adj, W_comb)
    return out


# speedup vs baseline: 1.3992x; 1.3992x over previous
import functools

import jax
import jax.numpy as jnp
from jax import lax
from jax.experimental import pallas as pl
from jax.experimental.pallas import tpu as pltpu
from jax.experimental.pallas import tpu_sc as plsc

N = 10000
E = 320000
D = 128
C = 500
CP = 512
BLK = 1000
GRID = N // BLK

NC = 2
NS = 16
DH = D // NC
EPS = E // NS
B = 125
NBS = EPS // B
NBUF = 2
NRING = NBS // NBUF
NPAD = 10240
RPT = NPAD // NS
RB = 80
RZ = RPT // RB


def _tc_pre_body(x_ref, cid_ref, wmsg_ref, b_ref,
                 pl_ref, ph_ref, ql_ref, qh_ref, cl_ref, cnt_ref):
    i = pl.program_id(0)
    xb = x_ref[...]
    w1 = wmsg_ref[0:D, :]
    w2 = wmsg_ref[D:2 * D, :]
    wp = w1 + 0.5 * w2
    wq = 0.5 * w2
    p = jnp.dot(xb, wp, preferred_element_type=jnp.float32) + b_ref[...]
    q = jnp.dot(xb, wq, preferred_element_type=jnp.float32)
    pl_ref[...] = p[:, 0:DH]
    ph_ref[...] = p[:, DH:D]
    ql_ref[...] = q[:, 0:DH]
    qh_ref[...] = q[:, DH:D]
    oh = (cid_ref[...] == lax.broadcasted_iota(jnp.int32, (BLK, CP), 1)).astype(jnp.float32)
    clb = lax.dot_general(oh, xb, (((0,), (0,)), ((), ())),
                          preferred_element_type=jnp.float32)
    cntb = lax.dot_general(oh, jnp.ones((BLK, D), jnp.float32), (((0,), (0,)), ((), ())),
                           preferred_element_type=jnp.float32)

    @pl.when(i == 0)
    def _():
        cl_ref[...] = jnp.zeros_like(cl_ref)
        cnt_ref[...] = jnp.zeros_like(cnt_ref)

    cl_ref[...] += clb
    cnt_ref[...] += cntb


_tc_pre = pl.pallas_call(
    _tc_pre_body,
    grid=(GRID,),
    in_specs=[
        pl.BlockSpec((BLK, D), lambda i: (i, 0)),
        pl.BlockSpec((BLK, 1), lambda i: (i, 0)),
        pl.BlockSpec((2 * D, D), lambda i: (0, 0)),
        pl.BlockSpec((1, D), lambda i: (0, 0)),
    ],
    out_specs=[
        pl.BlockSpec((BLK, DH), lambda i: (i, 0)),
        pl.BlockSpec((BLK, DH), lambda i: (i, 0)),
        pl.BlockSpec((BLK, DH), lambda i: (i, 0)),
        pl.BlockSpec((BLK, DH), lambda i: (i, 0)),
        pl.BlockSpec((CP, D), lambda i: (0, 0)),
        pl.BlockSpec((CP, D), lambda i: (0, 0)),
    ],
    out_shape=[
        jax.ShapeDtypeStruct((N, DH), jnp.float32),
        jax.ShapeDtypeStruct((N, DH), jnp.float32),
        jax.ShapeDtypeStruct((N, DH), jnp.float32),
        jax.ShapeDtypeStruct((N, DH), jnp.float32),
        jax.ShapeDtypeStruct((CP, D), jnp.float32),
        jax.ShapeDtypeStruct((CP, D), jnp.float32),
    ],
)


_sc_mesh = plsc.VectorSubcoreMesh(core_axis_name="c", subcore_axis_name="s")


@functools.partial(
    pl.kernel,
    mesh=_sc_mesh,
    out_type=jax.ShapeDtypeStruct((NC, NPAD, DH), jnp.float32),
    scratch_types=(
        [pltpu.VMEM((NBS, B), jnp.int32),
         pltpu.VMEM((NBS, B), jnp.int32)]
        + [pltpu.VMEM((B, DH), jnp.float32) for _ in range(2 * NBUF)]
        + [pltpu.VMEM((B, DH), jnp.float32) for _ in range(2)]
        + [pltpu.VMEM_SHARED((NPAD, DH), jnp.float32)]
        + [pltpu.SemaphoreType.DMA for _ in range(2 * NBUF + 2)]
    ),
    compiler_params=pltpu.CompilerParams(use_tc_tiling_on_sc=False),
)
def _sc_edge(pl_hbm, ph_hbm, ql_hbm, qh_hbm, src_hbm, dst_hbm, out_hbm,
             srcv, dstv,
             pb0, pb1, qb0, qb1, sb0, sb1, agg_sh,
             gp0, gp1, gq0, gq1, ss0, ss1):
    cid = lax.axis_index("c")
    sid = lax.axis_index("s")
    rb = sid * RPT
    pbs = (pb0, pb1)
    qbs = (qb0, qb1)
    sbs = (sb0, sb1)
    gps = (gp0, gp1)
    gqs = (gq0, gq1)
    sss = (ss0, ss1)

    pltpu.sync_copy(src_hbm.at[sid], srcv)
    pltpu.sync_copy(dst_hbm.at[sid], dstv)

    def _zrow(r, carry):
        for l in range(DH // 16):
            s = pl.ds(l * 16, 16)
            sb0[r, s] = jnp.zeros((16,), jnp.float32)
            sb1[r, s] = jnp.zeros((16,), jnp.float32)
        return carry

    lax.fori_loop(0, B, _zrow, 0)
    for z in range(RZ):
        pltpu.sync_copy(sb0.at[pl.ds(0, RB)], agg_sh.at[pl.ds(rb + z * RB, RB)])
    plsc.subcore_barrier()

    def _run(p_hbm, q_hbm):
        for k in range(2):
            pltpu.async_copy(sbs[k], agg_sh.at[dstv.at[k]], sss[k], add=True)
        for b in range(NBUF):
            pltpu.async_copy(p_hbm.at[srcv.at[b]], pbs[b], gps[b])
            pltpu.async_copy(q_hbm.at[dstv.at[b]], qbs[b], gqs[b])

        def _ring(g, carry):
            base = g * NBUF
            for b in range(NBUF):
                j = base + b
                k = b & 1
                pltpu.make_async_copy(p_hbm.at[srcv.at[j]], pbs[b], gps[b]).wait()
                pltpu.make_async_copy(q_hbm.at[dstv.at[j]], qbs[b], gqs[b]).wait()
                pltpu.make_async_copy(sbs[k], agg_sh.at[dstv.at[j]], sss[k]).wait()

                def _crow(r2, c2):
                    for u in range(5):
                        r = r2 * 5 + u
                        for l in range(DH // 16):
                            s = pl.ds(l * 16, 16)
                            sbs[k][r, s] = jnp.maximum(pbs[b][r, s] + qbs[b][r, s], 0.0)
                    return c2

                lax.fori_loop(0, B // 5, _crow, 0)
                pltpu.async_copy(sbs[k], agg_sh.at[dstv.at[j]], sss[k], add=True)
                j4 = j + NBUF
                jn = lax.select(j4 < NBS, j4, j4 - NBS)
                pltpu.async_copy(p_hbm.at[srcv.at[jn]], pbs[b], gps[b])
                pltpu.async_copy(q_hbm.at[dstv.at[jn]], qbs[b], gqs[b])
            return carry

        lax.fori_loop(0, NRING, _ring, 0)

        for k in range(2):
            pltpu.make_async_copy(sbs[k], agg_sh.at[dstv.at[k]], sss[k]).wait()
        for b in range(NBUF):
            pltpu.make_async_copy(p_hbm.at[srcv.at[b]], pbs[b], gps[b]).wait()
            pltpu.make_async_copy(q_hbm.at[dstv.at[b]], qbs[b], gqs[b]).wait()

    @pl.when(cid == 0)
    def _():
        _run(pl_hbm, ql_hbm)

    @pl.when(cid == 1)
    def _():
        _run(ph_hbm, qh_hbm)

    plsc.subcore_barrier()

    for z in range(RZ):
        pltpu.sync_copy(agg_sh.at[pl.ds(rb + z * RB, RB)], sb0.at[pl.ds(0, RB)])
        pltpu.sync_copy(sb0.at[pl.ds(0, RB)], out_hbm.at[cid, pl.ds(rb + z * RB, RB)])


def _tc_up_body(x_ref, cid_ref, cl_ref, cnt_ref, wup_ref, wu_ref, wc_ref, b_ref):
    xb = x_ref[...]
    up_attr = cl_ref[...] / jnp.maximum(cnt_ref[...], 1.0)
    msg_up = jnp.maximum(jnp.dot(up_attr, wup_ref[...], preferred_element_type=jnp.float32), 0.0)
    oh = (cid_ref[...] == lax.broadcasted_iota(jnp.int32, (BLK, CP), 1)).astype(jnp.float32)
    agg_up = jnp.dot(oh, msg_up, preferred_element_type=jnp.float32) + xb
    out_up = jnp.maximum(jnp.dot(agg_up, wu_ref[...], preferred_element_type=jnp.float32), 0.0)
    b_ref[...] = jnp.dot(out_up, wc_ref[D:2 * D, :], preferred_element_type=jnp.float32)


_tc_up = pl.pallas_call(
    _tc_up_body,
    grid=(GRID,),
    in_specs=[
        pl.BlockSpec((BLK, D), lambda i: (i, 0)),
        pl.BlockSpec((BLK, 1), lambda i: (i, 0)),
        pl.BlockSpec((CP, D), lambda i: (0, 0)),
        pl.BlockSpec((CP, D), lambda i: (0, 0)),
        pl.BlockSpec((D, D), lambda i: (0, 0)),
        pl.BlockSpec((D, D), lambda i: (0, 0)),
        pl.BlockSpec((2 * D, D), lambda i: (0, 0)),
    ],
    out_specs=pl.BlockSpec((BLK, D), lambda i: (i, 0)),
    out_shape=jax.ShapeDtypeStruct((N, D), jnp.float32),
)


def _tc_fin_body(x_ref, aggl_ref, aggh_ref, base_ref, wa_ref, wc_ref, o_ref):
    xb = x_ref[...]
    agg = jnp.concatenate([aggl_ref[...], aggh_ref[...]], axis=-1)
    adj = agg + xb
    out_adj = jnp.maximum(jnp.dot(adj, wa_ref[...], preferred_element_type=jnp.float32), 0.0)
    o_ref[...] = (jnp.dot(out_adj, wc_ref[0:D, :], preferred_element_type=jnp.float32)
                  + base_ref[...])


_tc_fin = pl.pallas_call(
    _tc_fin_body,
    grid=(GRID,),
    in_specs=[
        pl.BlockSpec((BLK, D), lambda i: (i, 0)),
        pl.BlockSpec((BLK, DH), lambda i: (i, 0)),
        pl.BlockSpec((BLK, DH), lambda i: (i, 0)),
        pl.BlockSpec((BLK, D), lambda i: (i, 0)),
        pl.BlockSpec((D, D), lambda i: (0, 0)),
        pl.BlockSpec((2 * D, D), lambda i: (0, 0)),
    ],
    out_specs=pl.BlockSpec((BLK, D), lambda i: (i, 0)),
    out_shape=jax.ShapeDtypeStruct((N, D), jnp.float32),
)


def kernel(x, edge_index, cluster_ids, W_msg, b_msg, W_up, W_upd_adj, W_upd_up, W_comb):
    src3 = edge_index[0].reshape(NS, NBS, B)
    dst3 = edge_index[1].reshape(NS, NBS, B)
    cid2 = cluster_ids.reshape(N, 1)
    bm = b_msg.reshape(1, D)
    PL, PH, QL, QH, cl, cnt = _tc_pre(x, cid2, W_msg, bm)
    parts = _sc_edge(PL, PH, QL, QH, src3, dst3)
    base = _tc_up(x, cid2, cl, cnt, W_up, W_upd_up, W_comb)
    out = _tc_fin(x, parts[0, :N], parts[1, :N], base, W_upd_adj, W_comb)
    return out


# speedup vs baseline: 1.5522x; 1.1093x over previous
import functools

import jax
import jax.numpy as jnp
from jax import lax
from jax.experimental import pallas as pl
from jax.experimental.pallas import tpu as pltpu
from jax.experimental.pallas import tpu_sc as plsc

N = 10000
E = 320000
D = 128
C = 500
CP = 512
BLK = 1000
GRID = N // BLK

NC = 2
NS = 16
DH = D // NC
EPS = E // NS
B = 125
NBS = EPS // B
NBUF = 2
NRING = NBS // NBUF
RPT = N // NS
RB = 125
RZ = RPT // RB


def _tc_pre_body(x_ref, cid_ref, wmsg_ref, b_ref,
                 pl_ref, ph_ref, ql_ref, qh_ref, cl_ref, cnt_ref):
    i = pl.program_id(0)
    xb = x_ref[...]
    w1 = wmsg_ref[0:D, :]
    w2 = wmsg_ref[D:2 * D, :]
    wp = w1 + 0.5 * w2
    wq = 0.5 * w2
    p = jnp.dot(xb, wp, preferred_element_type=jnp.float32) + b_ref[...]
    q = jnp.dot(xb, wq, preferred_element_type=jnp.float32)
    pl_ref[...] = p[:, 0:DH]
    ph_ref[...] = p[:, DH:D]
    ql_ref[...] = q[:, 0:DH]
    qh_ref[...] = q[:, DH:D]
    oh = (cid_ref[...] == lax.broadcasted_iota(jnp.int32, (BLK, CP), 1)).astype(jnp.float32)
    clb = lax.dot_general(oh, xb, (((0,), (0,)), ((), ())),
                          preferred_element_type=jnp.float32)
    cntb = lax.dot_general(oh, jnp.ones((BLK, D), jnp.float32), (((0,), (0,)), ((), ())),
                           preferred_element_type=jnp.float32)

    @pl.when(i == 0)
    def _():
        cl_ref[...] = jnp.zeros_like(cl_ref)
        cnt_ref[...] = jnp.zeros_like(cnt_ref)

    cl_ref[...] += clb
    cnt_ref[...] += cntb


_tc_pre = pl.pallas_call(
    _tc_pre_body,
    grid=(GRID,),
    in_specs=[
        pl.BlockSpec((BLK, D), lambda i: (i, 0)),
        pl.BlockSpec((BLK, 1), lambda i: (i, 0)),
        pl.BlockSpec((2 * D, D), lambda i: (0, 0)),
        pl.BlockSpec((1, D), lambda i: (0, 0)),
    ],
    out_specs=[
        pl.BlockSpec((BLK, DH), lambda i: (i, 0)),
        pl.BlockSpec((BLK, DH), lambda i: (i, 0)),
        pl.BlockSpec((BLK, DH), lambda i: (i, 0)),
        pl.BlockSpec((BLK, DH), lambda i: (i, 0)),
        pl.BlockSpec((CP, D), lambda i: (0, 0)),
        pl.BlockSpec((CP, D), lambda i: (0, 0)),
    ],
    out_shape=[
        jax.ShapeDtypeStruct((N, DH), jnp.float32),
        jax.ShapeDtypeStruct((N, DH), jnp.float32),
        jax.ShapeDtypeStruct((N, DH), jnp.float32),
        jax.ShapeDtypeStruct((N, DH), jnp.float32),
        jax.ShapeDtypeStruct((CP, D), jnp.float32),
        jax.ShapeDtypeStruct((CP, D), jnp.float32),
    ],
)


_sc_mesh = plsc.VectorSubcoreMesh(core_axis_name="c", subcore_axis_name="s")


@functools.partial(
    pl.kernel,
    mesh=_sc_mesh,
    out_type=jax.ShapeDtypeStruct((N, D), jnp.float32),
    scratch_types=(
        [pltpu.VMEM((NBS, B), jnp.int32),
         pltpu.VMEM((NBS, B), jnp.int32)]
        + [pltpu.VMEM((B, DH), jnp.float32) for _ in range(2 * NBUF)]
        + [pltpu.VMEM((B, DH), jnp.float32) for _ in range(2)]
        + [pltpu.VMEM_SHARED((N, DH), jnp.float32)]
        + [pltpu.SemaphoreType.DMA for _ in range(2 * NBUF + 2)]
    ),
    compiler_params=pltpu.CompilerParams(use_tc_tiling_on_sc=False),
)
def _sc_edge(pl_hbm, ph_hbm, ql_hbm, qh_hbm, e_hbm, out_hbm,
             srcv, dstv,
             pb0, pb1, qb0, qb1, sb0, sb1, agg_sh,
             gp0, gp1, gq0, gq1, ss0, ss1):
    cid = lax.axis_index("c")
    sid = lax.axis_index("s")
    rb = sid * RPT
    pbs = (pb0, pb1)
    qbs = (qb0, qb1)
    sbs = (sb0, sb1)
    gps = (gp0, gp1)
    gqs = (gq0, gq1)
    sss = (ss0, ss1)

    pltpu.sync_copy(e_hbm.at[0, sid], srcv)
    pltpu.sync_copy(e_hbm.at[1, sid], dstv)

    def _zrow(r, carry):
        for l in range(DH // 16):
            s = pl.ds(l * 16, 16)
            sb0[r, s] = jnp.zeros((16,), jnp.float32)
            sb1[r, s] = jnp.zeros((16,), jnp.float32)
        return carry

    lax.fori_loop(0, B, _zrow, 0)
    for z in range(RZ):
        pltpu.sync_copy(sb0.at[pl.ds(0, RB)], agg_sh.at[pl.ds(rb + z * RB, RB)])
    plsc.subcore_barrier()

    def _run(p_hbm, q_hbm):
        for k in range(2):
            pltpu.async_copy(sbs[k], agg_sh.at[dstv.at[k]], sss[k], add=True)
        for b in range(NBUF):
            pltpu.async_copy(p_hbm.at[srcv.at[b]], pbs[b], gps[b])
            pltpu.async_copy(q_hbm.at[dstv.at[b]], qbs[b], gqs[b])

        def _ring(g, carry):
            base = g * NBUF
            for b in range(NBUF):
                j = base + b
                k = b & 1
                pltpu.make_async_copy(p_hbm.at[srcv.at[j]], pbs[b], gps[b]).wait()
                pltpu.make_async_copy(q_hbm.at[dstv.at[j]], qbs[b], gqs[b]).wait()
                pltpu.make_async_copy(sbs[k], agg_sh.at[dstv.at[j]], sss[k]).wait()

                def _crow(r2, c2):
                    for u in range(5):
                        r = r2 * 5 + u
                        for l in range(DH // 16):
                            s = pl.ds(l * 16, 16)
                            sbs[k][r, s] = jnp.maximum(pbs[b][r, s] + qbs[b][r, s], 0.0)
                    return c2

                lax.fori_loop(0, B // 5, _crow, 0)
                pltpu.async_copy(sbs[k], agg_sh.at[dstv.at[j]], sss[k], add=True)
                j4 = j + NBUF
                jn = lax.select(j4 < NBS, j4, j4 - NBS)
                pltpu.async_copy(p_hbm.at[srcv.at[jn]], pbs[b], gps[b])
                pltpu.async_copy(q_hbm.at[dstv.at[jn]], qbs[b], gqs[b])
            return carry

        lax.fori_loop(0, NRING, _ring, 0)

        for k in range(2):
            pltpu.make_async_copy(sbs[k], agg_sh.at[dstv.at[k]], sss[k]).wait()
        for b in range(NBUF):
            pltpu.make_async_copy(p_hbm.at[srcv.at[b]], pbs[b], gps[b]).wait()
            pltpu.make_async_copy(q_hbm.at[dstv.at[b]], qbs[b], gqs[b]).wait()

    @pl.when(cid == 0)
    def _():
        _run(pl_hbm, ql_hbm)

    @pl.when(cid == 1)
    def _():
        _run(ph_hbm, qh_hbm)

    plsc.subcore_barrier()

    for z in range(RZ):
        pltpu.sync_copy(agg_sh.at[pl.ds(rb + z * RB, RB)], sb0.at[pl.ds(0, RB)])
        pltpu.sync_copy(sb0.at[pl.ds(0, RB)],
                        out_hbm.at[pl.ds(rb + z * RB, RB), pl.ds(cid * DH, DH)])


def _tc_up_body(x_ref, cid_ref, cl_ref, cnt_ref, wup_ref, wu_ref, wc_ref, b_ref):
    xb = x_ref[...]
    up_attr = cl_ref[...] / jnp.maximum(cnt_ref[...], 1.0)
    msg_up = jnp.maximum(jnp.dot(up_attr, wup_ref[...], preferred_element_type=jnp.float32), 0.0)
    oh = (cid_ref[...] == lax.broadcasted_iota(jnp.int32, (BLK, CP), 1)).astype(jnp.float32)
    agg_up = jnp.dot(oh, msg_up, preferred_element_type=jnp.float32) + xb
    out_up = jnp.maximum(jnp.dot(agg_up, wu_ref[...], preferred_element_type=jnp.float32), 0.0)
    b_ref[...] = jnp.dot(out_up, wc_ref[D:2 * D, :], preferred_element_type=jnp.float32)


_tc_up = pl.pallas_call(
    _tc_up_body,
    grid=(GRID,),
    in_specs=[
        pl.BlockSpec((BLK, D), lambda i: (i, 0)),
        pl.BlockSpec((BLK, 1), lambda i: (i, 0)),
        pl.BlockSpec((CP, D), lambda i: (0, 0)),
        pl.BlockSpec((CP, D), lambda i: (0, 0)),
        pl.BlockSpec((D, D), lambda i: (0, 0)),
        pl.BlockSpec((D, D), lambda i: (0, 0)),
        pl.BlockSpec((2 * D, D), lambda i: (0, 0)),
    ],
    out_specs=pl.BlockSpec((BLK, D), lambda i: (i, 0)),
    out_shape=jax.ShapeDtypeStruct((N, D), jnp.float32),
)


def _tc_fin_body(x_ref, agg_ref, base_ref, wa_ref, wc_ref, o_ref):
    xb = x_ref[...]
    adj = agg_ref[...] + xb
    out_adj = jnp.maximum(jnp.dot(adj, wa_ref[...], preferred_element_type=jnp.float32), 0.0)
    o_ref[...] = (jnp.dot(out_adj, wc_ref[0:D, :], preferred_element_type=jnp.float32)
                  + base_ref[...])


_tc_fin = pl.pallas_call(
    _tc_fin_body,
    grid=(GRID,),
    in_specs=[
        pl.BlockSpec((BLK, D), lambda i: (i, 0)),
        pl.BlockSpec((BLK, D), lambda i: (i, 0)),
        pl.BlockSpec((BLK, D), lambda i: (i, 0)),
        pl.BlockSpec((D, D), lambda i: (0, 0)),
        pl.BlockSpec((2 * D, D), lambda i: (0, 0)),
    ],
    out_specs=pl.BlockSpec((BLK, D), lambda i: (i, 0)),
    out_shape=jax.ShapeDtypeStruct((N, D), jnp.float32),
)


def kernel(x, edge_index, cluster_ids, W_msg, b_msg, W_up, W_upd_adj, W_upd_up, W_comb):
    edge4 = edge_index.reshape(2, NS, NBS, B)
    cid2 = cluster_ids.reshape(N, 1)
    bm = b_msg.reshape(1, D)
    PL, PH, QL, QH, cl, cnt = _tc_pre(x, cid2, W_msg, bm)
    agg = _sc_edge(PL, PH, QL, QH, edge4)
    base = _tc_up(x, cid2, cl, cnt, W_up, W_upd_up, W_comb)
    out = _tc_fin(x, agg, base, W_upd_adj, W_comb)
    return out


# speedup vs baseline: 1.6280x; 1.0488x over previous
import functools

import jax
import jax.numpy as jnp
from jax import lax
from jax.experimental import pallas as pl
from jax.experimental.pallas import tpu as pltpu
from jax.experimental.pallas import tpu_sc as plsc

N = 10000
E = 320000
D = 128
C = 500
CP = 512
BLK = 1000
GRID = N // BLK

NC = 2
NS = 16
DH = D // NC
EPS = E // NS
B = 125
NBS = EPS // B
NBUF = 2
NRING = NBS // NBUF
RPT = N // NS
RB = 125
RZ = RPT // RB


def _tc_pq_body(x_ref, wmsg_ref, b_ref, pl_ref, ph_ref, ql_ref, qh_ref):
    xb = x_ref[...]
    w1 = wmsg_ref[0:D, :]
    w2 = wmsg_ref[D:2 * D, :]
    wp = w1 + 0.5 * w2
    wq = 0.5 * w2
    p = jnp.dot(xb, wp, preferred_element_type=jnp.float32) + b_ref[...]
    q = jnp.dot(xb, wq, preferred_element_type=jnp.float32)
    pl_ref[...] = p[:, 0:DH]
    ph_ref[...] = p[:, DH:D]
    ql_ref[...] = q[:, 0:DH]
    qh_ref[...] = q[:, DH:D]


_tc_pq = pl.pallas_call(
    _tc_pq_body,
    grid=(GRID,),
    in_specs=[
        pl.BlockSpec((BLK, D), lambda i: (i, 0)),
        pl.BlockSpec((2 * D, D), lambda i: (0, 0)),
        pl.BlockSpec((1, D), lambda i: (0, 0)),
    ],
    out_specs=[
        pl.BlockSpec((BLK, DH), lambda i: (i, 0)),
        pl.BlockSpec((BLK, DH), lambda i: (i, 0)),
        pl.BlockSpec((BLK, DH), lambda i: (i, 0)),
        pl.BlockSpec((BLK, DH), lambda i: (i, 0)),
    ],
    out_shape=[
        jax.ShapeDtypeStruct((N, DH), jnp.float32),
        jax.ShapeDtypeStruct((N, DH), jnp.float32),
        jax.ShapeDtypeStruct((N, DH), jnp.float32),
        jax.ShapeDtypeStruct((N, DH), jnp.float32),
    ],
)


def _tc_cl_body(x_ref, cid_ref, cl_ref, cnt_ref):
    i = pl.program_id(0)
    xb = x_ref[...]
    oh = (cid_ref[...] == lax.broadcasted_iota(jnp.int32, (BLK, CP), 1)).astype(jnp.float32)
    clb = lax.dot_general(oh, xb, (((0,), (0,)), ((), ())),
                          preferred_element_type=jnp.float32)
    cntb = lax.dot_general(oh, jnp.ones((BLK, D), jnp.float32), (((0,), (0,)), ((), ())),
                           preferred_element_type=jnp.float32)

    @pl.when(i == 0)
    def _():
        cl_ref[...] = jnp.zeros_like(cl_ref)
        cnt_ref[...] = jnp.zeros_like(cnt_ref)

    cl_ref[...] += clb
    cnt_ref[...] += cntb


_tc_cl = pl.pallas_call(
    _tc_cl_body,
    grid=(GRID,),
    in_specs=[
        pl.BlockSpec((BLK, D), lambda i: (i, 0)),
        pl.BlockSpec((BLK, 1), lambda i: (i, 0)),
    ],
    out_specs=[
        pl.BlockSpec((CP, D), lambda i: (0, 0)),
        pl.BlockSpec((CP, D), lambda i: (0, 0)),
    ],
    out_shape=[
        jax.ShapeDtypeStruct((CP, D), jnp.float32),
        jax.ShapeDtypeStruct((CP, D), jnp.float32),
    ],
)


_sc_mesh = plsc.VectorSubcoreMesh(core_axis_name="c", subcore_axis_name="s")


@functools.partial(
    pl.kernel,
    mesh=_sc_mesh,
    out_type=jax.ShapeDtypeStruct((N, D), jnp.float32),
    scratch_types=(
        [pltpu.VMEM((NBS, B), jnp.int32),
         pltpu.VMEM((NBS, B), jnp.int32)]
        + [pltpu.VMEM((B, DH), jnp.float32) for _ in range(2 * NBUF)]
        + [pltpu.VMEM((B, DH), jnp.float32) for _ in range(2)]
        + [pltpu.VMEM_SHARED((N, DH), jnp.float32)]
        + [pltpu.SemaphoreType.DMA for _ in range(2 * NBUF + 2)]
    ),
    compiler_params=pltpu.CompilerParams(use_tc_tiling_on_sc=False),
)
def _sc_edge(pl_hbm, ph_hbm, ql_hbm, qh_hbm, e_hbm, out_hbm,
             srcv, dstv,
             pb0, pb1, qb0, qb1, sb0, sb1, agg_sh,
             gp0, gp1, gq0, gq1, ss0, ss1):
    cid = lax.axis_index("c")
    sid = lax.axis_index("s")
    rb = sid * RPT
    pbs = (pb0, pb1)
    qbs = (qb0, qb1)
    sbs = (sb0, sb1)
    gps = (gp0, gp1)
    gqs = (gq0, gq1)
    sss = (ss0, ss1)

    pltpu.sync_copy(e_hbm.at[0, sid], srcv)
    pltpu.sync_copy(e_hbm.at[1, sid], dstv)

    def _zrow(r, carry):
        for l in range(DH // 16):
            s = pl.ds(l * 16, 16)
            sb0[r, s] = jnp.zeros((16,), jnp.float32)
            sb1[r, s] = jnp.zeros((16,), jnp.float32)
        return carry

    lax.fori_loop(0, B, _zrow, 0)
    for z in range(RZ):
        pltpu.sync_copy(sb0.at[pl.ds(0, RB)], agg_sh.at[pl.ds(rb + z * RB, RB)])
    plsc.subcore_barrier()

    def _run(p_hbm, q_hbm):
        for k in range(2):
            pltpu.async_copy(sbs[k], agg_sh.at[dstv.at[k]], sss[k], add=True)
        for b in range(NBUF):
            pltpu.async_copy(p_hbm.at[srcv.at[b]], pbs[b], gps[b])
            pltpu.async_copy(q_hbm.at[dstv.at[b]], qbs[b], gqs[b])

        def _ring(g, carry):
            base = g * NBUF
            for b in range(NBUF):
                j = base + b
                k = b & 1
                pltpu.make_async_copy(p_hbm.at[srcv.at[j]], pbs[b], gps[b]).wait()
                pltpu.make_async_copy(q_hbm.at[dstv.at[j]], qbs[b], gqs[b]).wait()
                pltpu.make_async_copy(sbs[k], agg_sh.at[dstv.at[j]], sss[k]).wait()

                def _crow(r2, c2):
                    for u in range(5):
                        r = r2 * 5 + u
                        for l in range(DH // 16):
                            s = pl.ds(l * 16, 16)
                            sbs[k][r, s] = jnp.maximum(pbs[b][r, s] + qbs[b][r, s], 0.0)
                    return c2

                lax.fori_loop(0, B // 5, _crow, 0)
                pltpu.async_copy(sbs[k], agg_sh.at[dstv.at[j]], sss[k], add=True)
                j4 = j + NBUF
                jn = lax.select(j4 < NBS, j4, j4 - NBS)
                pltpu.async_copy(p_hbm.at[srcv.at[jn]], pbs[b], gps[b])
                pltpu.async_copy(q_hbm.at[dstv.at[jn]], qbs[b], gqs[b])
            return carry

        lax.fori_loop(0, NRING, _ring, 0)

        for k in range(2):
            pltpu.make_async_copy(sbs[k], agg_sh.at[dstv.at[k]], sss[k]).wait()
        for b in range(NBUF):
            pltpu.make_async_copy(p_hbm.at[srcv.at[b]], pbs[b], gps[b]).wait()
            pltpu.make_async_copy(q_hbm.at[dstv.at[b]], qbs[b], gqs[b]).wait()

    @pl.when(cid == 0)
    def _():
        _run(pl_hbm, ql_hbm)

    @pl.when(cid == 1)
    def _():
        _run(ph_hbm, qh_hbm)

    plsc.subcore_barrier()

    for z in range(RZ):
        pltpu.sync_copy(agg_sh.at[pl.ds(rb + z * RB, RB)], sb0.at[pl.ds(0, RB)])
        pltpu.sync_copy(sb0.at[pl.ds(0, RB)],
                        out_hbm.at[pl.ds(rb + z * RB, RB), pl.ds(cid * DH, DH)])


def _tc_up_body(x_ref, cid_ref, cl_ref, cnt_ref, wup_ref, wu_ref, wc_ref, b_ref):
    xb = x_ref[...]
    up_attr = cl_ref[...] / jnp.maximum(cnt_ref[...], 1.0)
    msg_up = jnp.maximum(jnp.dot(up_attr, wup_ref[...], preferred_element_type=jnp.float32), 0.0)
    oh = (cid_ref[...] == lax.broadcasted_iota(jnp.int32, (BLK, CP), 1)).astype(jnp.float32)
    agg_up = jnp.dot(oh, msg_up, preferred_element_type=jnp.float32) + xb
    out_up = jnp.maximum(jnp.dot(agg_up, wu_ref[...], preferred_element_type=jnp.float32), 0.0)
    b_ref[...] = jnp.dot(out_up, wc_ref[D:2 * D, :], preferred_element_type=jnp.float32)


_tc_up = pl.pallas_call(
    _tc_up_body,
    grid=(GRID,),
    in_specs=[
        pl.BlockSpec((BLK, D), lambda i: (i, 0)),
        pl.BlockSpec((BLK, 1), lambda i: (i, 0)),
        pl.BlockSpec((CP, D), lambda i: (0, 0)),
        pl.BlockSpec((CP, D), lambda i: (0, 0)),
        pl.BlockSpec((D, D), lambda i: (0, 0)),
        pl.BlockSpec((D, D), lambda i: (0, 0)),
        pl.BlockSpec((2 * D, D), lambda i: (0, 0)),
    ],
    out_specs=pl.BlockSpec((BLK, D), lambda i: (i, 0)),
    out_shape=jax.ShapeDtypeStruct((N, D), jnp.float32),
)


def _tc_fin_body(x_ref, agg_ref, base_ref, wa_ref, wc_ref, o_ref):
    xb = x_ref[...]
    adj = agg_ref[...] + xb
    out_adj = jnp.maximum(jnp.dot(adj, wa_ref[...], preferred_element_type=jnp.float32), 0.0)
    o_ref[...] = (jnp.dot(out_adj, wc_ref[0:D, :], preferred_element_type=jnp.float32)
                  + base_ref[...])


_tc_fin = pl.pallas_call(
    _tc_fin_body,
    grid=(GRID,),
    in_specs=[
        pl.BlockSpec((BLK, D), lambda i: (i, 0)),
        pl.BlockSpec((BLK, D), lambda i: (i, 0)),
        pl.BlockSpec((BLK, D), lambda i: (i, 0)),
        pl.BlockSpec((D, D), lambda i: (0, 0)),
        pl.BlockSpec((2 * D, D), lambda i: (0, 0)),
    ],
    out_specs=pl.BlockSpec((BLK, D), lambda i: (i, 0)),
    out_shape=jax.ShapeDtypeStruct((N, D), jnp.float32),
)


def kernel(x, edge_index, cluster_ids, W_msg, b_msg, W_up, W_upd_adj, W_upd_up, W_comb):
    edge4 = edge_index.reshape(2, NS, NBS, B)
    cid2 = cluster_ids.reshape(N, 1)
    bm = b_msg.reshape(1, D)
    PL, PH, QL, QH = _tc_pq(x, W_msg, bm)
    agg = _sc_edge(PL, PH, QL, QH, edge4)
    cl, cnt = _tc_cl(x, cid2)
    base = _tc_up(x, cid2, cl, cnt, W_up, W_upd_up, W_comb)
    out = _tc_fin(x, agg, base, W_upd_adj, W_comb)
    return out
